# baseline (speedup 1.0000x reference)
import functools

import jax
import jax.numpy as jnp
from jax import lax
from jax.experimental import pallas as pl
from jax.experimental.pallas import tpu as pltpu
from jax.experimental.pallas import tpu_sc as plsc

EMB = 64
ROW = 128
LANES = 16
CHUNK = 128


@functools.cache
def _sc_embed_dot(b_per_w, batch):
    n_chunks = b_per_w // CHUNK
    mesh = plsc.VectorSubcoreMesh(core_axis_name="c", subcore_axis_name="s")

    @functools.partial(
        pl.kernel,
        mesh=mesh,
        out_type=jax.ShapeDtypeStruct((batch,), jnp.float32),
        compiler_params=pltpu.CompilerParams(needs_layout_passes=False,
                                             use_tc_tiling_on_sc=True),
        scratch_types=[
            pltpu.VMEM((n_chunks, CHUNK), jnp.int32),
            pltpu.VMEM((n_chunks, CHUNK), jnp.int32),
            pltpu.VMEM((n_chunks, CHUNK), jnp.int32),
            pltpu.VMEM((n_chunks, CHUNK), jnp.int32),
            pltpu.VMEM((b_per_w,), jnp.int32),
            pltpu.VMEM((b_per_w,), jnp.int32),
            pltpu.VMEM((CHUNK, ROW), jnp.float32),
            pltpu.VMEM((CHUNK, ROW), jnp.float32),
            pltpu.VMEM((LANES,), jnp.float32),
            pltpu.VMEM((EMB * LANES,), jnp.float32),
            pltpu.VMEM((b_per_w,), jnp.float32),
            pltpu.SemaphoreType.DMA,
        ],
    )
    def sc_fn(user_hbm, item_hbm, ut_hbm, it_hbm, wb_hbm, b_hbm, out_hbm,
              uraw_v, iraw_v, uidx_v, iidx_v, upar_v, ipar_v,
              urows_v, irows_v, b_v, wb_v, out_v, sem):
        num_cores = 2
        wid = lax.axis_index("s") * num_cores + lax.axis_index("c")
        base = wid * b_per_w

        for j in range(n_chunks):
            off = base + j * CHUNK
            pltpu.sync_copy(user_hbm.at[pl.ds(off, CHUNK)], uraw_v.at[j])
            pltpu.sync_copy(item_hbm.at[pl.ds(off, CHUNK)], iraw_v.at[j])

        pltpu.sync_copy(wb_hbm, wb_v)
        pltpu.sync_copy(b_hbm, b_v)
        bias = b_v[...]
        lane_iota = lax.iota(jnp.int32, LANES)

        def prep_body(q, carry):
            sl = pl.ds(pl.multiple_of(q * LANES, LANES), LANES)
            j = q // (CHUNK // LANES)
            w = (q % (CHUNK // LANES)) * LANES
            slj = pl.ds(w, LANES)
            ru = uraw_v.at[j][slj]
            ri = iraw_v.at[j][slj]
            uidx_v.at[j][slj] = ru >> 1
            iidx_v.at[j][slj] = ri >> 1
            upar_v[sl] = (ru & 1) * EMB
            ipar_v[sl] = (ri & 1) * EMB
            return carry

        for q in range(b_per_w // LANES):
            prep_body(q, 0)

        for j in range(n_chunks):
            du = pltpu.async_copy(ut_hbm.at[uidx_v.at[j]], urows_v, sem)
            di = pltpu.async_copy(it_hbm.at[iidx_v.at[j]], irows_v, sem)
            du.wait()
            di.wait()

            def group_body(g, carry):
                row_idx = g * LANES + lane_iota
                off = pl.multiple_of(j * CHUNK, CHUNK) + g * LANES
                up = upar_v[pl.ds(off, LANES)]
                ip = ipar_v[pl.ds(off, LANES)]
                acc = bias
                for k in range(EMB):
                    gu = plsc.load_gather(urows_v, [row_idx, up + k])
                    gv = plsc.load_gather(irows_v, [row_idx, ip + k])
                    acc = acc + gu * gv * wb_v[pl.ds(k * LANES, LANES)]
                out_v[pl.ds(pl.multiple_of(j * CHUNK + g * LANES, LANES),
                            LANES)] = acc
                return carry

            lax.fori_loop(0, CHUNK // LANES, group_body, 0)

        pltpu.sync_copy(out_v, out_hbm.at[pl.ds(base, b_per_w)])

    return sc_fn


def kernel(user, item, user_table, item_table, W, b):
    batch = user.shape[0]
    num_workers = 32
    b_per_w = batch // num_workers
    nu = (user_table.shape[0] - 1) // 2
    ni = (item_table.shape[0] - 1) // 2
    ut2 = user_table[:2 * nu].reshape(nu, ROW)
    it2 = item_table[:2 * ni].reshape(ni, ROW)
    wb = jnp.broadcast_to(W.reshape(EMB, 1), (EMB, LANES)).reshape(-1)
    b16 = jnp.broadcast_to(b, (LANES,))
    fn = _sc_embed_dot(b_per_w, batch)
    out = fn(user, item, ut2, it2, wb, b16)
    return out.reshape(batch, 1)
